# Initial kernel scaffold; baseline (speedup 1.0000x reference)
import jax
import jax.numpy as jnp
from jax.experimental import pallas as pl


def kernel(x, neigh_orders, W, b):
    raise NotImplementedError("write your pallas kernel here")


# speedup vs baseline: 1.2081x; 1.2081x over previous
import functools

import jax
import jax.numpy as jnp
from jax import lax
from jax.experimental import pallas as pl
from jax.experimental.pallas import tpu as pltpu
from jax.experimental.pallas import tpu_sc as plsc

NC = 2
NS = 16
NW = NC * NS
P = 32
NB = 512
LANES = 16


def _y_matmul(x, w2):
    B, C, N = x.shape
    KO = w2.shape[1]
    nblk = pl.cdiv(N, NB)

    def body(x_ref, w_ref, y_ref):
        xb = x_ref[...].reshape(B * C, NB)
        y_ref[...] = lax.dot_general(
            xb, w_ref[...], (((0,), (0,)), ((), ())),
            preferred_element_type=jnp.float32)

    return pl.pallas_call(
        body,
        grid=(nblk,),
        in_specs=[
            pl.BlockSpec((B, C, NB), lambda i: (0, 0, i)),
            pl.BlockSpec((B * C, KO), lambda i: (0, 0)),
        ],
        out_specs=pl.BlockSpec((NB, KO), lambda i: (i, 0)),
        out_shape=jax.ShapeDtypeStruct((N, KO), jnp.float32),
    )(x, w2)


def _sc_gather_sum(y2, idx3, bias2, K, D, npad):
    T = npad // P
    G = T // NW

    mesh = plsc.VectorSubcoreMesh(
        core_axis_name="c", subcore_axis_name="s",
        num_cores=NC, num_subcores=NS)

    @functools.partial(
        pl.kernel,
        out_type=jax.ShapeDtypeStruct((npad, D), jnp.float32),
        mesh=mesh,
        scratch_types=[
            pltpu.VMEM((K, P), jnp.int32),
            pltpu.VMEM((K, P, D), jnp.float32),
            pltpu.VMEM((P, D), jnp.float32),
            pltpu.VMEM((D,), jnp.float32),
            pltpu.SemaphoreType.DMA,
        ],
    )
    def k(y2_hbm, idx_hbm, bias_hbm, out_hbm, idx_v, gath_v, acc_v, bias_v, sem):
        cid = lax.axis_index("c")
        sid = lax.axis_index("s")
        wid = sid * NC + cid
        pltpu.sync_copy(bias_hbm, bias_v)

        def chunk(g, carry):
            t = g * NW + wid
            base = t * P
            pltpu.sync_copy(idx_hbm.at[t], idx_v)
            handles = [
                pltpu.async_copy(y2_hbm.at[idx_v.at[kk]], gath_v.at[kk], sem)
                for kk in range(K)
            ]
            for h in handles:
                h.wait()

            def accum(p, c2):
                for cc in range(D // LANES):
                    sl = pl.ds(cc * LANES, LANES)
                    s = bias_v[sl]
                    for kk in range(K):
                        s = s + gath_v[kk, p, sl]
                    acc_v[p, sl] = s
                return c2

            lax.fori_loop(0, P, accum, 0)
            pltpu.sync_copy(acc_v, out_hbm.at[pl.ds(base, P)])
            return carry

        lax.fori_loop(0, G, chunk, 0)

    return k(y2, idx3, bias2)


def kernel(x, neigh_orders, W, b):
    B, C, N = x.shape
    K = neigh_orders.shape[1]
    OUT = W.shape[0]
    D = B * OUT

    wr = W.reshape(OUT, K, C).transpose(2, 1, 0)
    eyeb = jnp.eye(B, dtype=W.dtype)
    w2 = (wr[None, :, :, None, :] * eyeb[:, None, None, :, None]
          ).reshape(B * C, K * D)

    y2 = _y_matmul(x, w2).reshape(N * K, D)

    npad = ((N + NW * P - 1) // (NW * P)) * (NW * P)
    idx = neigh_orders.astype(jnp.int32) * K + jnp.arange(K, dtype=jnp.int32)[None, :]
    idx = jnp.pad(idx, ((0, npad - N), (0, 0)))
    idx3 = idx.reshape(npad // P, P, K).transpose(0, 2, 1)

    bias2 = jnp.tile(b, B)

    out = _sc_gather_sum(y2, idx3, bias2, K, D, npad)

    out = out[:N].reshape(N, B, OUT)
    return jnp.transpose(out, (1, 2, 0))
